# baseline (device time: 448683 ns/iter reference)
import jax
import jax.numpy as jnp
from jax import lax
from jax.experimental import pallas as pl
from jax.experimental.pallas import tpu as pltpu

Z = 4


def kernel(x, dest):
    t, d = x.shape
    dl = 128
    dr = t // dl
    dest2 = dest.reshape(dr, dl)

    def body(x_ref, d_ref, xall_ref, dall_ref,
             sx_send, sx_recv, sd_send, sd_recv):
        mx = lax.axis_index("x")
        my = lax.axis_index("y")
        mz = lax.axis_index("z")
        right = lax.rem(mz + 1, Z)
        left = lax.rem(mz + Z - 1, Z)

        barrier = pltpu.get_barrier_semaphore()
        for nbr in (left, right):
            pl.semaphore_signal(
                barrier, inc=1,
                device_id=(mx, my, nbr),
                device_id_type=pl.DeviceIdType.MESH,
            )
        pl.semaphore_wait(barrier, 2)

        xall_ref[pl.ds(mz * t, t), :] = x_ref[...]
        dall_ref[pl.ds(mz * dr, dr), :] = d_ref[...]

        for h in range(Z - 1):
            origin = lax.rem(mz - h + Z, Z)
            rx = pltpu.make_async_remote_copy(
                src_ref=xall_ref.at[pl.ds(origin * t, t), :],
                dst_ref=xall_ref.at[pl.ds(origin * t, t), :],
                send_sem=sx_send.at[h],
                recv_sem=sx_recv.at[h],
                device_id=(mx, my, right),
                device_id_type=pl.DeviceIdType.MESH,
            )
            rd = pltpu.make_async_remote_copy(
                src_ref=dall_ref.at[pl.ds(origin * dr, dr), :],
                dst_ref=dall_ref.at[pl.ds(origin * dr, dr), :],
                send_sem=sd_send.at[h],
                recv_sem=sd_recv.at[h],
                device_id=(mx, my, right),
                device_id_type=pl.DeviceIdType.MESH,
            )
            rx.start()
            rd.start()
            rx.wait()
            rd.wait()

    xall, dall = pl.pallas_call(
        body,
        out_shape=(
            jax.ShapeDtypeStruct((Z * t, d), jnp.float32),
            jax.ShapeDtypeStruct((Z * dr, dl), jnp.int32),
        ),
        in_specs=[
            pl.BlockSpec(memory_space=pltpu.VMEM),
            pl.BlockSpec(memory_space=pltpu.VMEM),
        ],
        out_specs=(
            pl.BlockSpec(memory_space=pltpu.VMEM),
            pl.BlockSpec(memory_space=pltpu.VMEM),
        ),
        scratch_shapes=[
            pltpu.SemaphoreType.DMA((Z - 1,)),
            pltpu.SemaphoreType.DMA((Z - 1,)),
            pltpu.SemaphoreType.DMA((Z - 1,)),
            pltpu.SemaphoreType.DMA((Z - 1,)),
        ],
        compiler_params=pltpu.CompilerParams(collective_id=0),
    )(x, dest2)

    dflat = dall.reshape(Z * t)
    order = jnp.argsort(dflat, stable=True)
    mz = lax.axis_index("z")
    idx = lax.dynamic_slice(order, (mz * t,), (t,))
    return jnp.take(xall, idx, axis=0)
